# baseline (speedup 1.0000x reference)
import functools

import jax
import jax.numpy as jnp
from jax import lax
from jax.experimental import pallas as pl
from jax.experimental.pallas import tpu as pltpu
from jax.experimental.pallas import tpu_sc as plsc

_N = 10000
_E = 320000
_D = 128
_EPS = 1e-5

_NC = 2
_NS = 16
_NW = _NC * _NS

_CHUNK = 128
_NCHUNK = 80
_DEPTH = 2
_NQ = 4
_EPT = _CHUNK * _NCHUNK
_EPAD = _EPT * _NW
_SHIFT = 14
_MASK = (1 << _SHIFT) - 1

_NP = 10112
_ROWS_PER_SUB = _NP // _NS
_PAD_ROW = 10015

_NPD = 10240
_DEG_PER_SUB = _NPD // _NS
_DCHUNK = 64
_DNCHUNK = 160
_DEPT = _DCHUNK * _DNCHUNK
_DEPAD = _DEPT * _NW

_mesh = plsc.VectorSubcoreMesh(core_axis_name="c", subcore_axis_name="s")


def _unpack(pk_v, j, sidx, srow, didx, drow):
    base = j * _CHUNK
    for i in range(_CHUNK // 16):
        p = pk_v[pl.ds(base + i * 16, 16)]
        sidx[srow, pl.ds(i * 16, 16)] = lax.shift_right_logical(p, _SHIFT)
        didx[drow, pl.ds(i * 16, 16)] = lax.bitwise_and(p, _MASK)


@functools.partial(
    pl.kernel,
    out_type=jax.ShapeDtypeStruct((_NC, _NPD), jnp.float32),
    mesh=_mesh,
    scratch_types=[
        pltpu.VMEM_SHARED((_NPD,), jnp.float32),
        pltpu.VMEM((_DNCHUNK // 2, _DCHUNK), jnp.int32),
        pltpu.VMEM((_DCHUNK,), jnp.float32),
    ],
)
def _sc_degree(dstp_hbm, zeros_hbm, ones_hbm, out_hbm, dacc, dst_v, ones_v):
    cid = lax.axis_index("c")
    sid = lax.axis_index("s")
    wid = sid * _NC + cid
    half = _DNCHUNK // 2

    pltpu.sync_copy(ones_hbm, ones_v)
    pltpu.sync_copy(zeros_hbm, dacc.at[pl.ds(sid * _DEG_PER_SUB, _DEG_PER_SUB)])
    plsc.subcore_barrier()

    def chunk(j, carry):
        pltpu.sync_copy(ones_v, dacc.at[dst_v.at[j]], add=True)
        return carry

    for h in range(2):
        pltpu.sync_copy(dstp_hbm.at[wid].at[pl.ds(h * half, half)], dst_v)
        lax.fori_loop(0, half, chunk, 0)
    plsc.subcore_barrier()
    pltpu.sync_copy(
        dacc.at[pl.ds(sid * _DEG_PER_SUB, _DEG_PER_SUB)],
        out_hbm.at[cid].at[pl.ds(sid * _DEG_PER_SUB, _DEG_PER_SUB)],
    )


@functools.partial(
    pl.kernel,
    out_type=jax.ShapeDtypeStruct((_NC, _NP, _D), jnp.float32),
    mesh=_mesh,
    scratch_types=[
        pltpu.VMEM_SHARED((_NP, _D), jnp.float32),
        pltpu.VMEM((_EPT // _NQ,), jnp.int32),
        pltpu.VMEM((_DEPTH, _CHUNK), jnp.int32),
        pltpu.VMEM((_DEPTH, _CHUNK), jnp.int32),
    ]
    + [pltpu.VMEM((_CHUNK, _D), jnp.float32)] * _DEPTH
    + [pltpu.SemaphoreType.DMA] * _DEPTH,
)
def _sc_scatter(u_hbm, pk_hbm, zeros_hbm, out_hbm, acc, pk_v, sidx, didx, *ring):
    rows = ring[:_DEPTH]
    gsem = ring[_DEPTH:]
    cid = lax.axis_index("c")
    sid = lax.axis_index("s")
    wid = sid * _NC + cid
    qchunk = _NCHUNK // _NQ

    pltpu.sync_copy(zeros_hbm, acc.at[pl.ds(sid * _ROWS_PER_SUB, _ROWS_PER_SUB)])
    plsc.subcore_barrier()

    def gather(b, j):
        _unpack(pk_v, j, sidx, b, didx, b)
        pltpu.make_async_copy(u_hbm.at[sidx.at[b]], rows[b], gsem[b]).start()

    def drain(b):
        pltpu.make_async_copy(u_hbm.at[sidx.at[b]], rows[b], gsem[b]).wait()
        pltpu.sync_copy(rows[b], acc.at[didx.at[b]], add=True)

    for q in range(_NQ):
        pltpu.sync_copy(
            pk_hbm.at[wid].at[pl.ds(q * qchunk * _CHUNK, qchunk * _CHUNK)], pk_v
        )
        for b in range(_DEPTH):
            gather(b, b)

        def group(step, carry):
            for b in range(_DEPTH):
                drain(b)
                gather(b, step * _DEPTH + b + _DEPTH)
            return carry

        lax.fori_loop(0, qchunk // _DEPTH - 1, group, 0)
        for b in range(_DEPTH):
            drain(b)
    plsc.subcore_barrier()
    pltpu.sync_copy(
        acc.at[pl.ds(sid * _ROWS_PER_SUB, _ROWS_PER_SUB)],
        out_hbm.at[cid].at[pl.ds(sid * _ROWS_PER_SUB, _ROWS_PER_SUB)],
    )


_BLK = 1000
_NBLK = _N // _BLK

_row_spec = pl.BlockSpec((_BLK, _D), lambda i: (i, 0))
_col_spec = pl.BlockSpec((_BLK, 1), lambda i: (i, 0))
_w_spec = pl.BlockSpec((_D, _D), lambda i: (0, 0))
_v_spec = pl.BlockSpec((1, _D), lambda i: (0, 0))
_s_spec = pl.BlockSpec((_NC, _BLK, _D), lambda i: (0, i, 0))


def _tc_in_body(x_ref, w_ref, dinv_ref, o_ref):
    o_ref[...] = jnp.dot(dinv_ref[...] * x_ref[...], w_ref[...],
                         preferred_element_type=jnp.float32)


_tc_in = pl.pallas_call(
    _tc_in_body,
    grid=(_NBLK,),
    in_specs=[_row_spec, _w_spec, _col_spec],
    out_specs=_row_spec,
    out_shape=jax.ShapeDtypeStruct((_N, _D), jnp.float32),
)


def _tc_mid_body(s_ref, u_ref, dinv_ref, b_ref, g_ref, be_ref, w_ref, o_ref):
    dinv = dinv_ref[...]
    t = s_ref[0] + s_ref[1] + u_ref[...]
    z = dinv * t + b_ref[...]
    y = jnp.maximum(z * g_ref[...] + be_ref[...], 0.0)
    o_ref[...] = jnp.dot(dinv * y, w_ref[...], preferred_element_type=jnp.float32)


_tc_mid = pl.pallas_call(
    _tc_mid_body,
    grid=(_NBLK,),
    in_specs=[_s_spec, _row_spec, _col_spec, _v_spec, _v_spec, _v_spec, _w_spec],
    out_specs=_row_spec,
    out_shape=jax.ShapeDtypeStruct((_N, _D), jnp.float32),
)


def _tc_out_body(s_ref, u_ref, dinv_ref, b_ref, g_ref, be_ref, w_ref, rob_ref, o_ref):
    t = s_ref[0] + s_ref[1] + u_ref[...]
    z = dinv_ref[...] * t + b_ref[...]
    y = jnp.maximum(z * g_ref[...] + be_ref[...], 0.0)
    o_ref[...] = jnp.dot(y, w_ref[...], preferred_element_type=jnp.float32) + rob_ref[...]


_tc_out = pl.pallas_call(
    _tc_out_body,
    grid=(_NBLK,),
    in_specs=[_s_spec, _row_spec, _col_spec, _v_spec, _v_spec, _v_spec, _w_spec, _v_spec],
    out_specs=_row_spec,
    out_shape=jax.ShapeDtypeStruct((_N, _D), jnp.float32),
)


def kernel(x, edge_index, W0, b0, g0, be0, W1, b1, g1, be1, W2, b2, g2, be2, roW, rob):
    src = edge_index[0]
    dst = edge_index[1]
    pad = _EPAD - _E
    packed = jnp.concatenate(
        [(src << _SHIFT) | dst, jnp.full((pad,), _PAD_ROW, jnp.int32)]
    ).reshape(_NW, _EPT)
    dstp = jnp.concatenate(
        [dst, jnp.full((_DEPAD - _E,), _PAD_ROW, jnp.int32)]
    ).reshape(_NW, _DNCHUNK, _DCHUNK)

    zeros_deg = jnp.zeros((_DEG_PER_SUB,), jnp.float32)
    ones_deg = jnp.ones((_DCHUNK,), jnp.float32)
    zeros_acc = jnp.zeros((_ROWS_PER_SUB, _D), jnp.float32)

    degp = _sc_degree(dstp, zeros_deg, ones_deg)
    deg = degp[0, :_N] + degp[1, :_N] + 1.0
    dinv = (deg ** -0.5).reshape(_N, 1)

    bn_scale = 1.0 / jnp.sqrt(1.0 + _EPS)
    row = lambda v: v.reshape(1, _D)
    g0s, g1s, g2s = row(g0) * bn_scale, row(g1) * bn_scale, row(g2) * bn_scale

    u = _tc_in(x, W0, dinv)
    s = _sc_scatter(u, packed, zeros_acc)
    u = _tc_mid(s, u, dinv, row(b0), g0s, row(be0), W1)
    s = _sc_scatter(u, packed, zeros_acc)
    u = _tc_mid(s, u, dinv, row(b1), g1s, row(be1), W2)
    s = _sc_scatter(u, packed, zeros_acc)
    return _tc_out(s, u, dinv, row(b2), g2s, row(be2), roW, row(rob))


# speedup vs baseline: 1.0401x; 1.0401x over previous
import functools

import jax
import jax.numpy as jnp
from jax import lax
from jax.experimental import pallas as pl
from jax.experimental.pallas import tpu as pltpu
from jax.experimental.pallas import tpu_sc as plsc

_N = 10000
_E = 320000
_D = 128
_EPS = 1e-5

_NC = 2
_NS = 16
_NW = _NC * _NS

_CHUNK = 128
_NCHUNK = 80
_DEPTH = 2
_EPT = _CHUNK * _NCHUNK
_EPAD = _EPT * _NW
_SHIFT = 14
_MASK = (1 << _SHIFT) - 1

_NP = 10112
_ROWS_PER_SUB = _NP // _NS
_PAD_ROW = 10015

_NPD = 10240
_DEG_PER_SUB = _NPD // _NS
_DCHUNK = 64
_DNCHUNK = 160
_DEPT = _DCHUNK * _DNCHUNK
_DEPAD = _DEPT * _NW

_mesh = plsc.VectorSubcoreMesh(core_axis_name="c", subcore_axis_name="s")


def _unpack(pkbuf, row, sidx, didx):
    for i in range(_CHUNK // 16):
        p = pkbuf[row, pl.ds(i * 16, 16)]
        sidx[row, pl.ds(i * 16, 16)] = lax.shift_right_logical(p, _SHIFT)
        didx[row, pl.ds(i * 16, 16)] = lax.bitwise_and(p, _MASK)


@functools.partial(
    pl.kernel,
    out_type=jax.ShapeDtypeStruct((_NC, _NPD), jnp.float32),
    mesh=_mesh,
    scratch_types=[
        pltpu.VMEM_SHARED((_NPD,), jnp.float32),
        pltpu.VMEM((_DNCHUNK // 2, _DCHUNK), jnp.int32),
        pltpu.VMEM((_DCHUNK,), jnp.float32),
    ],
)
def _sc_degree(dstp_hbm, zeros_hbm, ones_hbm, out_hbm, dacc, dst_v, ones_v):
    cid = lax.axis_index("c")
    sid = lax.axis_index("s")
    wid = sid * _NC + cid
    half = _DNCHUNK // 2

    pltpu.sync_copy(ones_hbm, ones_v)
    pltpu.sync_copy(zeros_hbm, dacc.at[pl.ds(sid * _DEG_PER_SUB, _DEG_PER_SUB)])
    plsc.subcore_barrier()

    def chunk(j, carry):
        pltpu.sync_copy(ones_v, dacc.at[dst_v.at[j]], add=True)
        return carry

    for h in range(2):
        pltpu.sync_copy(dstp_hbm.at[wid].at[pl.ds(h * half, half)], dst_v)
        lax.fori_loop(0, half, chunk, 0)
    plsc.subcore_barrier()
    pltpu.sync_copy(
        dacc.at[pl.ds(sid * _DEG_PER_SUB, _DEG_PER_SUB)],
        out_hbm.at[cid].at[pl.ds(sid * _DEG_PER_SUB, _DEG_PER_SUB)],
    )


@functools.partial(
    pl.kernel,
    out_type=jax.ShapeDtypeStruct((_NC, _NP, _D), jnp.float32),
    mesh=_mesh,
    scratch_types=[
        pltpu.VMEM_SHARED((_NP, _D), jnp.float32),
        pltpu.VMEM((_DEPTH, _CHUNK), jnp.int32),
        pltpu.VMEM((_DEPTH, _CHUNK), jnp.int32),
        pltpu.VMEM((_DEPTH, _CHUNK), jnp.int32),
    ]
    + [pltpu.VMEM((_CHUNK, _D), jnp.float32)] * _DEPTH
    + [pltpu.SemaphoreType.DMA] * (2 * _DEPTH),
)
def _sc_scatter(u_hbm, pk_hbm, zeros_hbm, out_hbm, acc, pkbuf, sidx, didx, *ring):
    rows = ring[:_DEPTH]
    gsem = ring[_DEPTH:2 * _DEPTH]
    psem = ring[2 * _DEPTH:]
    cid = lax.axis_index("c")
    sid = lax.axis_index("s")
    wid = sid * _NC + cid
    pk_t = pk_hbm.at[wid]

    pltpu.sync_copy(zeros_hbm, acc.at[pl.ds(sid * _ROWS_PER_SUB, _ROWS_PER_SUB)])

    def pkfetch(b, j):
        jm = lax.rem(j, _NCHUNK) if isinstance(j, jax.Array) else j % _NCHUNK
        pltpu.make_async_copy(
            pk_t.at[pl.ds(jm * _CHUNK, _CHUNK)], pkbuf.at[b], psem[b]
        ).start()

    def gather(b, j):
        pltpu.make_async_copy(
            pk_t.at[pl.ds(0, _CHUNK)], pkbuf.at[b], psem[b]
        ).wait()
        _unpack(pkbuf, b, sidx, didx)
        pltpu.make_async_copy(u_hbm.at[sidx.at[b]], rows[b], gsem[b]).start()
        pkfetch(b, j + _DEPTH)

    def drain(b):
        pltpu.make_async_copy(u_hbm.at[sidx.at[b]], rows[b], gsem[b]).wait()
        pltpu.sync_copy(rows[b], acc.at[didx.at[b]], add=True)

    for b in range(_DEPTH):
        pkfetch(b, b)
    plsc.subcore_barrier()
    for b in range(_DEPTH):
        gather(b, b)

    def group(step, carry):
        for b in range(_DEPTH):
            drain(b)
            gather(b, step * _DEPTH + b + _DEPTH)
        return carry

    lax.fori_loop(0, _NCHUNK // _DEPTH - 1, group, 0)
    for b in range(_DEPTH):
        drain(b)
        pltpu.make_async_copy(
            pk_t.at[pl.ds(0, _CHUNK)], pkbuf.at[b], psem[b]
        ).wait()
    plsc.subcore_barrier()
    pltpu.sync_copy(
        acc.at[pl.ds(sid * _ROWS_PER_SUB, _ROWS_PER_SUB)],
        out_hbm.at[cid].at[pl.ds(sid * _ROWS_PER_SUB, _ROWS_PER_SUB)],
    )


_BLK = 1000
_NBLK = _N // _BLK

_row_spec = pl.BlockSpec((_BLK, _D), lambda i: (i, 0))
_col_spec = pl.BlockSpec((_BLK, 1), lambda i: (i, 0))
_w_spec = pl.BlockSpec((_D, _D), lambda i: (0, 0))
_v_spec = pl.BlockSpec((1, _D), lambda i: (0, 0))
_s_spec = pl.BlockSpec((_NC, _BLK, _D), lambda i: (0, i, 0))


def _tc_in_body(x_ref, w_ref, dinv_ref, o_ref):
    o_ref[...] = jnp.dot(dinv_ref[...] * x_ref[...], w_ref[...],
                         preferred_element_type=jnp.float32)


_tc_in = pl.pallas_call(
    _tc_in_body,
    grid=(_NBLK,),
    in_specs=[_row_spec, _w_spec, _col_spec],
    out_specs=_row_spec,
    out_shape=jax.ShapeDtypeStruct((_N, _D), jnp.float32),
)


def _tc_mid_body(s_ref, u_ref, dinv_ref, b_ref, g_ref, be_ref, w_ref, o_ref):
    dinv = dinv_ref[...]
    t = s_ref[0] + s_ref[1] + u_ref[...]
    z = dinv * t + b_ref[...]
    y = jnp.maximum(z * g_ref[...] + be_ref[...], 0.0)
    o_ref[...] = jnp.dot(dinv * y, w_ref[...], preferred_element_type=jnp.float32)


_tc_mid = pl.pallas_call(
    _tc_mid_body,
    grid=(_NBLK,),
    in_specs=[_s_spec, _row_spec, _col_spec, _v_spec, _v_spec, _v_spec, _w_spec],
    out_specs=_row_spec,
    out_shape=jax.ShapeDtypeStruct((_N, _D), jnp.float32),
)


def _tc_out_body(s_ref, u_ref, dinv_ref, b_ref, g_ref, be_ref, w_ref, rob_ref, o_ref):
    t = s_ref[0] + s_ref[1] + u_ref[...]
    z = dinv_ref[...] * t + b_ref[...]
    y = jnp.maximum(z * g_ref[...] + be_ref[...], 0.0)
    o_ref[...] = jnp.dot(y, w_ref[...], preferred_element_type=jnp.float32) + rob_ref[...]


_tc_out = pl.pallas_call(
    _tc_out_body,
    grid=(_NBLK,),
    in_specs=[_s_spec, _row_spec, _col_spec, _v_spec, _v_spec, _v_spec, _w_spec, _v_spec],
    out_specs=_row_spec,
    out_shape=jax.ShapeDtypeStruct((_N, _D), jnp.float32),
)


def kernel(x, edge_index, W0, b0, g0, be0, W1, b1, g1, be1, W2, b2, g2, be2, roW, rob):
    src = edge_index[0]
    dst = edge_index[1]
    pad = _EPAD - _E
    packed = jnp.concatenate(
        [(src << _SHIFT) | dst, jnp.full((pad,), _PAD_ROW, jnp.int32)]
    ).reshape(_NW, _EPT)
    dstp = jnp.concatenate(
        [dst, jnp.full((_DEPAD - _E,), _PAD_ROW, jnp.int32)]
    ).reshape(_NW, _DNCHUNK, _DCHUNK)

    zeros_deg = jnp.zeros((_DEG_PER_SUB,), jnp.float32)
    ones_deg = jnp.ones((_DCHUNK,), jnp.float32)
    zeros_acc = jnp.zeros((_ROWS_PER_SUB, _D), jnp.float32)

    degp = _sc_degree(dstp, zeros_deg, ones_deg)
    deg = degp[0, :_N] + degp[1, :_N] + 1.0
    dinv = (deg ** -0.5).reshape(_N, 1)

    bn_scale = 1.0 / jnp.sqrt(1.0 + _EPS)
    row = lambda v: v.reshape(1, _D)
    g0s, g1s, g2s = row(g0) * bn_scale, row(g1) * bn_scale, row(g2) * bn_scale

    u = _tc_in(x, W0, dinv)
    s = _sc_scatter(u, packed, zeros_acc)
    u = _tc_mid(s, u, dinv, row(b0), g0s, row(be0), W1)
    s = _sc_scatter(u, packed, zeros_acc)
    u = _tc_mid(s, u, dinv, row(b1), g1s, row(be1), W2)
    s = _sc_scatter(u, packed, zeros_acc)
    return _tc_out(s, u, dinv, row(b2), g2s, row(be2), roW, row(rob))


# speedup vs baseline: 1.1781x; 1.1327x over previous
import functools

import jax
import jax.numpy as jnp
from jax import lax
from jax.experimental import pallas as pl
from jax.experimental.pallas import tpu as pltpu
from jax.experimental.pallas import tpu_sc as plsc

_N = 10000
_E = 320000
_D = 128
_EPS = 1e-5

_NC = 2
_NS = 16
_NW = _NC * _NS

_CHUNK = 96
_NCHUNK = 106
_DEPTH = 2
_EPT = _CHUNK * _NCHUNK
_EPAD = _EPT * _NW
_SHIFT = 14
_MASK = (1 << _SHIFT) - 1

_NP = 10112
_ROWS_PER_SUB = _NP // _NS
_PAD_ROW = 10015

_NPD = 10240
_DEG_PER_SUB = _NPD // _NS
_DCHUNK = 64
_DNCHUNK = 160
_DEPT = _DCHUNK * _DNCHUNK
_DEPAD = _DEPT * _NW

_mesh = plsc.VectorSubcoreMesh(core_axis_name="c", subcore_axis_name="s")


def _unpack(pkbuf, row, sidx, didx):
    for i in range(_CHUNK // 16):
        p = pkbuf[row, pl.ds(i * 16, 16)]
        sidx[row, pl.ds(i * 16, 16)] = lax.shift_right_logical(p, _SHIFT)
        didx[row, pl.ds(i * 16, 16)] = lax.bitwise_and(p, _MASK)


@functools.partial(
    pl.kernel,
    out_type=jax.ShapeDtypeStruct((_NC, _NPD), jnp.float32),
    mesh=_mesh,
    scratch_types=[
        pltpu.VMEM_SHARED((_NPD,), jnp.float32),
        pltpu.VMEM((_DNCHUNK // 2, _DCHUNK), jnp.int32),
        pltpu.VMEM((_DCHUNK,), jnp.float32),
    ],
)
def _sc_degree(dstp_hbm, zeros_hbm, ones_hbm, out_hbm, dacc, dst_v, ones_v):
    cid = lax.axis_index("c")
    sid = lax.axis_index("s")
    wid = sid * _NC + cid
    half = _DNCHUNK // 2

    pltpu.sync_copy(ones_hbm, ones_v)
    pltpu.sync_copy(zeros_hbm, dacc.at[pl.ds(sid * _DEG_PER_SUB, _DEG_PER_SUB)])
    plsc.subcore_barrier()

    def chunk(j, carry):
        pltpu.sync_copy(ones_v, dacc.at[dst_v.at[j]], add=True)
        return carry

    for h in range(2):
        pltpu.sync_copy(dstp_hbm.at[wid].at[pl.ds(h * half, half)], dst_v)
        lax.fori_loop(0, half, chunk, 0)
    plsc.subcore_barrier()
    pltpu.sync_copy(
        dacc.at[pl.ds(sid * _DEG_PER_SUB, _DEG_PER_SUB)],
        out_hbm.at[cid].at[pl.ds(sid * _DEG_PER_SUB, _DEG_PER_SUB)],
    )


@functools.partial(
    pl.kernel,
    out_type=jax.ShapeDtypeStruct((_NC, _NP, _D), jnp.float32),
    mesh=_mesh,
    scratch_types=[
        pltpu.VMEM_SHARED((_NP, _D), jnp.float32),
        pltpu.VMEM((_DEPTH, 128), jnp.int32),
        pltpu.VMEM((_DEPTH, _CHUNK), jnp.int32),
        pltpu.VMEM((_DEPTH, _CHUNK), jnp.int32),
    ]
    + [pltpu.VMEM((_CHUNK, _D), jnp.float32)] * _DEPTH
    + [pltpu.SemaphoreType.DMA] * (2 * _DEPTH),
)
def _sc_scatter(u_hbm, pk_hbm, zeros_hbm, out_hbm, acc, pkbuf, sidx, didx, *ring):
    rows = ring[:_DEPTH]
    gsem = ring[_DEPTH:2 * _DEPTH]
    psem = ring[2 * _DEPTH:]
    cid = lax.axis_index("c")
    sid = lax.axis_index("s")
    wid = sid * _NC + cid
    pk_t = pk_hbm.at[wid]

    pltpu.sync_copy(zeros_hbm, acc.at[pl.ds(sid * _ROWS_PER_SUB, _ROWS_PER_SUB)])

    def pkfetch(b, j):
        jm = lax.rem(j, _NCHUNK) if isinstance(j, jax.Array) else j % _NCHUNK
        pltpu.make_async_copy(
            pk_t.at[pl.ds(jm * 128, 128)], pkbuf.at[b], psem[b]
        ).start()

    def gather(b, j):
        pltpu.make_async_copy(
            pk_t.at[pl.ds(0, 128)], pkbuf.at[b], psem[b]
        ).wait()
        _unpack(pkbuf, b, sidx, didx)
        pltpu.make_async_copy(u_hbm.at[sidx.at[b]], rows[b], gsem[b]).start()
        pkfetch(b, j + _DEPTH)

    def drain(b):
        pltpu.make_async_copy(u_hbm.at[sidx.at[b]], rows[b], gsem[b]).wait()
        pltpu.sync_copy(rows[b], acc.at[didx.at[b]], add=True)

    for b in range(_DEPTH):
        pkfetch(b, b)
    plsc.subcore_barrier()
    for b in range(_DEPTH):
        gather(b, b)

    def group(step, carry):
        for b in range(_DEPTH):
            drain(b)
            gather(b, step * _DEPTH + b + _DEPTH)
        return carry

    lax.fori_loop(0, _NCHUNK // _DEPTH - 1, group, 0)
    for b in range(_DEPTH):
        drain(b)
        pltpu.make_async_copy(
            pk_t.at[pl.ds(0, 128)], pkbuf.at[b], psem[b]
        ).wait()
    plsc.subcore_barrier()
    pltpu.sync_copy(
        acc.at[pl.ds(sid * _ROWS_PER_SUB, _ROWS_PER_SUB)],
        out_hbm.at[cid].at[pl.ds(sid * _ROWS_PER_SUB, _ROWS_PER_SUB)],
    )


_BLK = 1000
_NBLK = _N // _BLK

_row_spec = pl.BlockSpec((_BLK, _D), lambda i: (i, 0))
_col_spec = pl.BlockSpec((_BLK, 1), lambda i: (i, 0))
_w_spec = pl.BlockSpec((_D, _D), lambda i: (0, 0))
_v_spec = pl.BlockSpec((1, _D), lambda i: (0, 0))
_s_spec = pl.BlockSpec((_NC, _BLK, _D), lambda i: (0, i, 0))


def _tc_in_body(x_ref, w_ref, dinv_ref, o_ref):
    o_ref[...] = jnp.dot(dinv_ref[...] * x_ref[...], w_ref[...],
                         preferred_element_type=jnp.float32)


_tc_in = pl.pallas_call(
    _tc_in_body,
    grid=(_NBLK,),
    in_specs=[_row_spec, _w_spec, _col_spec],
    out_specs=_row_spec,
    out_shape=jax.ShapeDtypeStruct((_N, _D), jnp.float32),
)


def _tc_mid_body(s_ref, u_ref, dinv_ref, b_ref, g_ref, be_ref, w_ref, o_ref):
    dinv = dinv_ref[...]
    t = s_ref[0] + s_ref[1] + u_ref[...]
    z = dinv * t + b_ref[...]
    y = jnp.maximum(z * g_ref[...] + be_ref[...], 0.0)
    o_ref[...] = jnp.dot(dinv * y, w_ref[...], preferred_element_type=jnp.float32)


_tc_mid = pl.pallas_call(
    _tc_mid_body,
    grid=(_NBLK,),
    in_specs=[_s_spec, _row_spec, _col_spec, _v_spec, _v_spec, _v_spec, _w_spec],
    out_specs=_row_spec,
    out_shape=jax.ShapeDtypeStruct((_N, _D), jnp.float32),
)


def _tc_out_body(s_ref, u_ref, dinv_ref, b_ref, g_ref, be_ref, w_ref, rob_ref, o_ref):
    t = s_ref[0] + s_ref[1] + u_ref[...]
    z = dinv_ref[...] * t + b_ref[...]
    y = jnp.maximum(z * g_ref[...] + be_ref[...], 0.0)
    o_ref[...] = jnp.dot(y, w_ref[...], preferred_element_type=jnp.float32) + rob_ref[...]


_tc_out = pl.pallas_call(
    _tc_out_body,
    grid=(_NBLK,),
    in_specs=[_s_spec, _row_spec, _col_spec, _v_spec, _v_spec, _v_spec, _w_spec, _v_spec],
    out_specs=_row_spec,
    out_shape=jax.ShapeDtypeStruct((_N, _D), jnp.float32),
)


def kernel(x, edge_index, W0, b0, g0, be0, W1, b1, g1, be1, W2, b2, g2, be2, roW, rob):
    src = edge_index[0]
    dst = edge_index[1]
    pad = _EPAD - _E
    packed = jnp.concatenate(
        [(src << _SHIFT) | dst, jnp.full((pad,), _PAD_ROW, jnp.int32)]
    ).reshape(_NW, _NCHUNK, _CHUNK)
    packed = jnp.pad(packed, ((0, 0), (0, 0), (0, 128 - _CHUNK))).reshape(
        _NW, _NCHUNK * 128
    )
    dstp = jnp.concatenate(
        [dst, jnp.full((_DEPAD - _E,), _PAD_ROW, jnp.int32)]
    ).reshape(_NW, _DNCHUNK, _DCHUNK)

    zeros_deg = jnp.zeros((_DEG_PER_SUB,), jnp.float32)
    ones_deg = jnp.ones((_DCHUNK,), jnp.float32)
    zeros_acc = jnp.zeros((_ROWS_PER_SUB, _D), jnp.float32)

    degp = _sc_degree(dstp, zeros_deg, ones_deg)
    deg = degp[0, :_N] + degp[1, :_N] + 1.0
    dinv = (deg ** -0.5).reshape(_N, 1)

    bn_scale = 1.0 / jnp.sqrt(1.0 + _EPS)
    row = lambda v: v.reshape(1, _D)
    g0s, g1s, g2s = row(g0) * bn_scale, row(g1) * bn_scale, row(g2) * bn_scale

    u = _tc_in(x, W0, dinv)
    s = _sc_scatter(u, packed, zeros_acc)
    u = _tc_mid(s, u, dinv, row(b0), g0s, row(be0), W1)
    s = _sc_scatter(u, packed, zeros_acc)
    u = _tc_mid(s, u, dinv, row(b1), g1s, row(be1), W2)
    s = _sc_scatter(u, packed, zeros_acc)
    return _tc_out(s, u, dinv, row(b2), g2s, row(be2), roW, row(rob))


# speedup vs baseline: 1.8782x; 1.5943x over previous
import functools

import jax
import jax.numpy as jnp
from jax import lax
from jax.experimental import pallas as pl
from jax.experimental.pallas import tpu as pltpu
from jax.experimental.pallas import tpu_sc as plsc

_N = 10000
_E = 320000
_D = 128
_EPS = 1e-5

_NC = 2
_NS = 16
_NW = _NC * _NS

_CHUNK = 96
_NCHUNK = 105
_EPT = _CHUNK * _NCHUNK
_EPAD = _EPT * _NW
_SHIFT = 14
_MASK = (1 << _SHIFT) - 1

_NP = 10112
_ROWS_PER_SUB = _NP // _NS
_PAD_ROW = 10015

_NPD = 10240
_DEG_PER_SUB = _NPD // _NS
_DCHUNK = 64
_DNCHUNK = 160
_DEPT = _DCHUNK * _DNCHUNK
_DEPAD = _DEPT * _NW

_mesh = plsc.VectorSubcoreMesh(core_axis_name="c", subcore_axis_name="s")


def _unpack(pk_v, j, sidx, didx, row):
    for i in range(_CHUNK // 16):
        p = pk_v[j, pl.ds(i * 16, 16)]
        sidx[row, pl.ds(i * 16, 16)] = lax.shift_right_logical(p, _SHIFT)
        didx[row, pl.ds(i * 16, 16)] = lax.bitwise_and(p, _MASK)


@functools.partial(
    pl.kernel,
    out_type=jax.ShapeDtypeStruct((_NC, _NPD), jnp.float32),
    mesh=_mesh,
    scratch_types=[
        pltpu.VMEM_SHARED((_NPD,), jnp.float32),
        pltpu.VMEM((_DNCHUNK // 2, _DCHUNK), jnp.int32),
        pltpu.VMEM((_DCHUNK,), jnp.float32),
    ],
)
def _sc_degree(dstp_hbm, zeros_hbm, ones_hbm, out_hbm, dacc, dst_v, ones_v):
    cid = lax.axis_index("c")
    sid = lax.axis_index("s")
    wid = sid * _NC + cid
    half = _DNCHUNK // 2

    pltpu.sync_copy(ones_hbm, ones_v)
    pltpu.sync_copy(zeros_hbm, dacc.at[pl.ds(sid * _DEG_PER_SUB, _DEG_PER_SUB)])
    plsc.subcore_barrier()

    def chunk(j, carry):
        pltpu.sync_copy(ones_v, dacc.at[dst_v.at[j]], add=True)
        return carry

    for h in range(2):
        pltpu.sync_copy(dstp_hbm.at[wid].at[pl.ds(h * half, half)], dst_v)
        lax.fori_loop(0, half, chunk, 0)
    plsc.subcore_barrier()
    pltpu.sync_copy(
        dacc.at[pl.ds(sid * _DEG_PER_SUB, _DEG_PER_SUB)],
        out_hbm.at[cid].at[pl.ds(sid * _DEG_PER_SUB, _DEG_PER_SUB)],
    )


@functools.partial(
    pl.kernel,
    out_type=jax.ShapeDtypeStruct((_NC, _NP, _D), jnp.float32),
    mesh=_mesh,
    scratch_types=[
        pltpu.VMEM_SHARED((_NP, _D), jnp.float32),
        pltpu.VMEM((_NCHUNK, _CHUNK), jnp.int32),
        pltpu.VMEM((2, _CHUNK), jnp.int32),
        pltpu.VMEM((2, _CHUNK), jnp.int32),
        pltpu.VMEM((_CHUNK, _D), jnp.float32),
        pltpu.VMEM((_CHUNK, _D), jnp.float32),
        pltpu.SemaphoreType.DMA,
        pltpu.SemaphoreType.DMA,
    ],
)
def _sc_scatter(u_hbm, pk_hbm, zeros_hbm, out_hbm,
                acc, pk_v, sidx, didx, rows0, rows1, sem0, sem1):
    cid = lax.axis_index("c")
    sid = lax.axis_index("s")
    wid = sid * _NC + cid

    pltpu.sync_copy(pk_hbm.at[wid], pk_v)
    pltpu.sync_copy(zeros_hbm, acc.at[pl.ds(sid * _ROWS_PER_SUB, _ROWS_PER_SUB)])
    plsc.subcore_barrier()

    def gather(j, b, buf, sem):
        _unpack(pk_v, j, sidx, didx, b)
        pltpu.make_async_copy(u_hbm.at[sidx.at[b]], buf, sem).start()

    def wait(b, buf, sem):
        pltpu.make_async_copy(u_hbm.at[sidx.at[b]], buf, sem).wait()

    gather(0, 0, rows0, sem0)

    def pair(s, carry):
        j1 = 2 * s + 1
        gather(j1, 1, rows1, sem1)
        wait(0, rows0, sem0)
        pltpu.sync_copy(rows0, acc.at[didx.at[0]], add=True)
        gather(j1 + 1, 0, rows0, sem0)
        wait(1, rows1, sem1)
        pltpu.sync_copy(rows1, acc.at[didx.at[1]], add=True)
        return carry

    lax.fori_loop(0, (_NCHUNK - 1) // 2, pair, 0)
    wait(0, rows0, sem0)
    pltpu.sync_copy(rows0, acc.at[didx.at[0]], add=True)
    plsc.subcore_barrier()
    pltpu.sync_copy(
        acc.at[pl.ds(sid * _ROWS_PER_SUB, _ROWS_PER_SUB)],
        out_hbm.at[cid].at[pl.ds(sid * _ROWS_PER_SUB, _ROWS_PER_SUB)],
    )


_BLK = 1000
_NBLK = _N // _BLK

_row_spec = pl.BlockSpec((_BLK, _D), lambda i: (i, 0))
_col_spec = pl.BlockSpec((_BLK, 1), lambda i: (i, 0))
_w_spec = pl.BlockSpec((_D, _D), lambda i: (0, 0))
_v_spec = pl.BlockSpec((1, _D), lambda i: (0, 0))
_s_spec = pl.BlockSpec((_NC, _BLK, _D), lambda i: (0, i, 0))


def _tc_in_body(x_ref, w_ref, dinv_ref, o_ref):
    o_ref[...] = jnp.dot(dinv_ref[...] * x_ref[...], w_ref[...],
                         preferred_element_type=jnp.float32)


_tc_in = pl.pallas_call(
    _tc_in_body,
    grid=(_NBLK,),
    in_specs=[_row_spec, _w_spec, _col_spec],
    out_specs=_row_spec,
    out_shape=jax.ShapeDtypeStruct((_N, _D), jnp.float32),
)


def _tc_mid_body(s_ref, u_ref, dinv_ref, b_ref, g_ref, be_ref, w_ref, o_ref):
    dinv = dinv_ref[...]
    t = s_ref[0] + s_ref[1] + u_ref[...]
    z = dinv * t + b_ref[...]
    y = jnp.maximum(z * g_ref[...] + be_ref[...], 0.0)
    o_ref[...] = jnp.dot(dinv * y, w_ref[...], preferred_element_type=jnp.float32)


_tc_mid = pl.pallas_call(
    _tc_mid_body,
    grid=(_NBLK,),
    in_specs=[_s_spec, _row_spec, _col_spec, _v_spec, _v_spec, _v_spec, _w_spec],
    out_specs=_row_spec,
    out_shape=jax.ShapeDtypeStruct((_N, _D), jnp.float32),
)


def _tc_out_body(s_ref, u_ref, dinv_ref, b_ref, g_ref, be_ref, w_ref, rob_ref, o_ref):
    t = s_ref[0] + s_ref[1] + u_ref[...]
    z = dinv_ref[...] * t + b_ref[...]
    y = jnp.maximum(z * g_ref[...] + be_ref[...], 0.0)
    o_ref[...] = jnp.dot(y, w_ref[...], preferred_element_type=jnp.float32) + rob_ref[...]


_tc_out = pl.pallas_call(
    _tc_out_body,
    grid=(_NBLK,),
    in_specs=[_s_spec, _row_spec, _col_spec, _v_spec, _v_spec, _v_spec, _w_spec, _v_spec],
    out_specs=_row_spec,
    out_shape=jax.ShapeDtypeStruct((_N, _D), jnp.float32),
)


def kernel(x, edge_index, W0, b0, g0, be0, W1, b1, g1, be1, W2, b2, g2, be2, roW, rob):
    src = edge_index[0]
    dst = edge_index[1]
    pad = _EPAD - _E
    packed = jnp.concatenate(
        [(src << _SHIFT) | dst, jnp.full((pad,), _PAD_ROW, jnp.int32)]
    ).reshape(_NW, _NCHUNK, _CHUNK)
    dstp = jnp.concatenate(
        [dst, jnp.full((_DEPAD - _E,), _PAD_ROW, jnp.int32)]
    ).reshape(_NW, _DNCHUNK, _DCHUNK)

    zeros_deg = jnp.zeros((_DEG_PER_SUB,), jnp.float32)
    ones_deg = jnp.ones((_DCHUNK,), jnp.float32)
    zeros_acc = jnp.zeros((_ROWS_PER_SUB, _D), jnp.float32)

    degp = _sc_degree(dstp, zeros_deg, ones_deg)
    deg = degp[0, :_N] + degp[1, :_N] + 1.0
    dinv = (deg ** -0.5).reshape(_N, 1)

    bn_scale = 1.0 / jnp.sqrt(1.0 + _EPS)
    row = lambda v: v.reshape(1, _D)
    g0s, g1s, g2s = row(g0) * bn_scale, row(g1) * bn_scale, row(g2) * bn_scale

    u = _tc_in(x, W0, dinv)
    s = _sc_scatter(u, packed, zeros_acc)
    u = _tc_mid(s, u, dinv, row(b0), g0s, row(be0), W1)
    s = _sc_scatter(u, packed, zeros_acc)
    u = _tc_mid(s, u, dinv, row(b1), g1s, row(be1), W2)
    s = _sc_scatter(u, packed, zeros_acc)
    return _tc_out(s, u, dinv, row(b2), g2s, row(be2), roW, row(rob))


# speedup vs baseline: 1.9659x; 1.0467x over previous
import functools

import jax
import jax.numpy as jnp
from jax import lax
from jax.experimental import pallas as pl
from jax.experimental.pallas import tpu as pltpu
from jax.experimental.pallas import tpu_sc as plsc

_N = 10000
_E = 320000
_D = 128
_EPS = 1e-5

_NC = 2
_NS = 16
_NW = _NC * _NS

_CHUNK = 80
_NCH0 = 165
_NCH1 = 87
_NCHMAX = _NCH0
_EPAD = (_NCH0 + _NCH1) * _CHUNK * _NS
_SHIFT = 14
_MASK = (1 << _SHIFT) - 1

_NP = 10112
_ROWS_PER_SUB = _NP // _NS
_PAD_ROW = 10015

_NPD = 10240
_DEG_PER_SUB = _NPD // _NS
_DCHUNK = 64
_DNCHUNK = 160
_DEPT = _DCHUNK * _DNCHUNK
_DEPAD = _DEPT * _NW

_mesh = plsc.VectorSubcoreMesh(core_axis_name="c", subcore_axis_name="s")


def _unpack(pk_v, j, sidx, didx, row):
    for i in range(_CHUNK // 16):
        p = pk_v[j, pl.ds(i * 16, 16)]
        sidx[row, pl.ds(i * 16, 16)] = lax.shift_right_logical(p, _SHIFT)
        didx[row, pl.ds(i * 16, 16)] = lax.bitwise_and(p, _MASK)


@functools.partial(
    pl.kernel,
    out_type=jax.ShapeDtypeStruct((_NC, _NPD), jnp.float32),
    mesh=_mesh,
    scratch_types=[
        pltpu.VMEM_SHARED((_NPD,), jnp.float32),
        pltpu.VMEM((_DNCHUNK // 2, _DCHUNK), jnp.int32),
        pltpu.VMEM((_DCHUNK,), jnp.float32),
    ],
)
def _sc_degree(dstp_hbm, zeros_hbm, ones_hbm, out_hbm, dacc, dst_v, ones_v):
    cid = lax.axis_index("c")
    sid = lax.axis_index("s")
    wid = sid * _NC + cid
    half = _DNCHUNK // 2

    pltpu.sync_copy(ones_hbm, ones_v)
    pltpu.sync_copy(zeros_hbm, dacc.at[pl.ds(sid * _DEG_PER_SUB, _DEG_PER_SUB)])
    plsc.subcore_barrier()

    def chunk(j, carry):
        pltpu.sync_copy(ones_v, dacc.at[dst_v.at[j]], add=True)
        return carry

    for h in range(2):
        pltpu.sync_copy(dstp_hbm.at[wid].at[pl.ds(h * half, half)], dst_v)
        lax.fori_loop(0, half, chunk, 0)
    plsc.subcore_barrier()
    pltpu.sync_copy(
        dacc.at[pl.ds(sid * _DEG_PER_SUB, _DEG_PER_SUB)],
        out_hbm.at[cid].at[pl.ds(sid * _DEG_PER_SUB, _DEG_PER_SUB)],
    )


@functools.partial(
    pl.kernel,
    out_type=jax.ShapeDtypeStruct((_NC, _NP, _D), jnp.float32),
    mesh=_mesh,
    scratch_types=[
        pltpu.VMEM_SHARED((_NP, _D), jnp.float32),
        pltpu.VMEM((_NCHMAX, _CHUNK), jnp.int32),
        pltpu.VMEM((2, _CHUNK), jnp.int32),
        pltpu.VMEM((2, _CHUNK), jnp.int32),
        pltpu.VMEM((_CHUNK, _D), jnp.float32),
        pltpu.VMEM((_CHUNK, _D), jnp.float32),
        pltpu.SemaphoreType.DMA,
        pltpu.SemaphoreType.DMA,
    ],
)
def _sc_scatter(u_hbm, pk_hbm, zeros_hbm, out_hbm,
                acc, pk_v, sidx, didx, rows0, rows1, sem0, sem1):
    cid = lax.axis_index("c")
    sid = lax.axis_index("s")
    wid = sid * _NC + cid

    pltpu.sync_copy(pk_hbm.at[wid], pk_v)
    pltpu.sync_copy(zeros_hbm, acc.at[pl.ds(sid * _ROWS_PER_SUB, _ROWS_PER_SUB)])
    plsc.subcore_barrier()

    def gather(j, b, buf, sem):
        _unpack(pk_v, j, sidx, didx, b)
        pltpu.make_async_copy(u_hbm.at[sidx.at[b]], buf, sem).start()

    def wait(b, buf, sem):
        pltpu.make_async_copy(u_hbm.at[sidx.at[b]], buf, sem).wait()

    npairs = lax.select(cid == 0, (_NCH0 - 1) // 2, (_NCH1 - 1) // 2)
    gather(0, 0, rows0, sem0)

    def pair(s, carry):
        j1 = 2 * s + 1
        gather(j1, 1, rows1, sem1)
        wait(0, rows0, sem0)
        pltpu.sync_copy(rows0, acc.at[didx.at[0]], add=True)
        gather(j1 + 1, 0, rows0, sem0)
        wait(1, rows1, sem1)
        pltpu.sync_copy(rows1, acc.at[didx.at[1]], add=True)
        return carry

    lax.fori_loop(0, npairs, pair, 0)
    wait(0, rows0, sem0)
    pltpu.sync_copy(rows0, acc.at[didx.at[0]], add=True)
    plsc.subcore_barrier()
    pltpu.sync_copy(
        acc.at[pl.ds(sid * _ROWS_PER_SUB, _ROWS_PER_SUB)],
        out_hbm.at[cid].at[pl.ds(sid * _ROWS_PER_SUB, _ROWS_PER_SUB)],
    )


_BLK = 1000
_NBLK = _N // _BLK

_row_spec = pl.BlockSpec((_BLK, _D), lambda i: (i, 0))
_col_spec = pl.BlockSpec((_BLK, 1), lambda i: (i, 0))
_w_spec = pl.BlockSpec((_D, _D), lambda i: (0, 0))
_v_spec = pl.BlockSpec((1, _D), lambda i: (0, 0))
_s_spec = pl.BlockSpec((_NC, _BLK, _D), lambda i: (0, i, 0))


def _tc_in_body(x_ref, w_ref, dinv_ref, o_ref):
    o_ref[...] = jnp.dot(dinv_ref[...] * x_ref[...], w_ref[...],
                         preferred_element_type=jnp.float32)


_tc_in = pl.pallas_call(
    _tc_in_body,
    grid=(_NBLK,),
    in_specs=[_row_spec, _w_spec, _col_spec],
    out_specs=_row_spec,
    out_shape=jax.ShapeDtypeStruct((_N, _D), jnp.float32),
)


def _tc_mid_body(s_ref, u_ref, dinv_ref, b_ref, g_ref, be_ref, w_ref, o_ref):
    dinv = dinv_ref[...]
    t = s_ref[0] + s_ref[1] + u_ref[...]
    z = dinv * t + b_ref[...]
    y = jnp.maximum(z * g_ref[...] + be_ref[...], 0.0)
    o_ref[...] = jnp.dot(dinv * y, w_ref[...], preferred_element_type=jnp.float32)


_tc_mid = pl.pallas_call(
    _tc_mid_body,
    grid=(_NBLK,),
    in_specs=[_s_spec, _row_spec, _col_spec, _v_spec, _v_spec, _v_spec, _w_spec],
    out_specs=_row_spec,
    out_shape=jax.ShapeDtypeStruct((_N, _D), jnp.float32),
)


def _tc_out_body(s_ref, u_ref, dinv_ref, b_ref, g_ref, be_ref, w_ref, rob_ref, o_ref):
    t = s_ref[0] + s_ref[1] + u_ref[...]
    z = dinv_ref[...] * t + b_ref[...]
    y = jnp.maximum(z * g_ref[...] + be_ref[...], 0.0)
    o_ref[...] = jnp.dot(y, w_ref[...], preferred_element_type=jnp.float32) + rob_ref[...]


_tc_out = pl.pallas_call(
    _tc_out_body,
    grid=(_NBLK,),
    in_specs=[_s_spec, _row_spec, _col_spec, _v_spec, _v_spec, _v_spec, _w_spec, _v_spec],
    out_specs=_row_spec,
    out_shape=jax.ShapeDtypeStruct((_N, _D), jnp.float32),
)


def kernel(x, edge_index, W0, b0, g0, be0, W1, b1, g1, be1, W2, b2, g2, be2, roW, rob):
    src = edge_index[0]
    dst = edge_index[1]
    pad = _EPAD - _E
    flat = jnp.concatenate(
        [(src << _SHIFT) | dst, jnp.full((pad,), _PAD_ROW, jnp.int32)]
    )
    n0 = _NS * _NCH0 * _CHUNK
    ea = flat[:n0].reshape(_NS, _NCH0, _CHUNK)
    eb = flat[n0:].reshape(_NS, _NCH1, _CHUNK)
    eb = jnp.pad(eb, ((0, 0), (0, _NCH0 - _NCH1), (0, 0)),
                 constant_values=_PAD_ROW)
    packed = jnp.stack([ea, eb], axis=1).reshape(_NW, _NCHMAX, _CHUNK)
    dstp = jnp.concatenate(
        [dst, jnp.full((_DEPAD - _E,), _PAD_ROW, jnp.int32)]
    ).reshape(_NW, _DNCHUNK, _DCHUNK)

    zeros_deg = jnp.zeros((_DEG_PER_SUB,), jnp.float32)
    ones_deg = jnp.ones((_DCHUNK,), jnp.float32)
    zeros_acc = jnp.zeros((_ROWS_PER_SUB, _D), jnp.float32)

    degp = _sc_degree(dstp, zeros_deg, ones_deg)
    deg = degp[0, :_N] + degp[1, :_N] + 1.0
    dinv = (deg ** -0.5).reshape(_N, 1)

    bn_scale = 1.0 / jnp.sqrt(1.0 + _EPS)
    row = lambda v: v.reshape(1, _D)
    g0s, g1s, g2s = row(g0) * bn_scale, row(g1) * bn_scale, row(g2) * bn_scale

    u = _tc_in(x, W0, dinv)
    s = _sc_scatter(u, packed, zeros_acc)
    u = _tc_mid(s, u, dinv, row(b0), g0s, row(be0), W1)
    s = _sc_scatter(u, packed, zeros_acc)
    u = _tc_mid(s, u, dinv, row(b1), g1s, row(be1), W2)
    s = _sc_scatter(u, packed, zeros_acc)
    return _tc_out(s, u, dinv, row(b2), g2s, row(be2), roW, row(rob))
